# Initial kernel scaffold; baseline (speedup 1.0000x reference)
import jax
import jax.numpy as jnp
from jax.experimental import pallas as pl


def kernel(x):
    raise NotImplementedError("write your pallas kernel here")


# speedup vs baseline: 3.7203x; 3.7203x over previous
import jax
import jax.numpy as jnp
from jax.experimental import pallas as pl


_NC = 16


def _upsample_body(x_ref, o_ref):
    val = x_ref[...]
    nc, h, w = val.shape
    col = jax.lax.broadcasted_iota(jnp.int32, (w, 2 * w), 1)
    row = jax.lax.broadcasted_iota(jnp.int32, (w, 2 * w), 0)
    scat = (col == 2 * row + 1).astype(val.dtype)
    right = jax.lax.dot_general(
        val.reshape(nc * h, w), scat,
        dimension_numbers=(((1,), (0,)), ((), ())),
        preferred_element_type=val.dtype,
        precision=jax.lax.Precision.HIGHEST,
    ).reshape(nc, h, 2 * w)
    o_ref[:, :, : 2 * w] = jnp.zeros((nc, h, 2 * w), val.dtype)
    o_ref[:, :, 2 * w :] = right


def kernel(x):
    B, I, C, H, W = x.shape
    n = B * I * C
    xf = x.reshape(n, H, W)
    out = pl.pallas_call(
        _upsample_body,
        grid=(n // _NC,),
        in_specs=[pl.BlockSpec((_NC, H, W), lambda i: (i, 0, 0))],
        out_specs=pl.BlockSpec((_NC, H, 4 * W), lambda i: (i, 0, 0)),
        out_shape=jax.ShapeDtypeStruct((n, H, 4 * W), x.dtype),
    )(xf)
    return out.reshape(B, I, C, 2 * H, 2 * W)


# speedup vs baseline: 3.9290x; 1.0561x over previous
import jax
import jax.numpy as jnp
from jax.experimental import pallas as pl


_NC = 32


def _upsample_body(x_ref, o_ref):
    val = x_ref[...]
    nc, h, w = val.shape
    col = jax.lax.broadcasted_iota(jnp.int32, (w, 2 * w), 1)
    row = jax.lax.broadcasted_iota(jnp.int32, (w, 2 * w), 0)
    scat = (col == 2 * row + 1).astype(val.dtype)
    right = jax.lax.dot_general(
        val.reshape(nc * h, w), scat,
        dimension_numbers=(((1,), (0,)), ((), ())),
        preferred_element_type=val.dtype,
        precision=jax.lax.Precision.HIGHEST,
    ).reshape(nc, h, 2 * w)
    o_ref[:, :, : 2 * w] = jnp.zeros((nc, h, 2 * w), val.dtype)
    o_ref[:, :, 2 * w :] = right


def kernel(x):
    B, I, C, H, W = x.shape
    n = B * I * C
    xf = x.reshape(n, H, W)
    out = pl.pallas_call(
        _upsample_body,
        grid=(n // _NC,),
        in_specs=[pl.BlockSpec((_NC, H, W), lambda i: (i, 0, 0))],
        out_specs=pl.BlockSpec((_NC, H, 4 * W), lambda i: (i, 0, 0)),
        out_shape=jax.ShapeDtypeStruct((n, H, 4 * W), x.dtype),
    )(xf)
    return out.reshape(B, I, C, 2 * H, 2 * W)
